# baseline (speedup 1.0000x reference)
import functools

import jax
import jax.numpy as jnp
from jax import lax
from jax.experimental import pallas as pl
from jax.experimental.pallas import tpu as pltpu
from jax.experimental.pallas import tpu_sc as plsc

N_NODES = 10000
N_EDGES = 160000
D = 256

NC, NS = 2, 16
NW = NC * NS
RPW = 320
NPAD = NW * RPW
ECH = 3200
NCHUNK = N_EDGES // ECH
NSL = D // 16
SELCAP = ECH + 144


def _matmul_body(f_ref, w_ref, o_ref):
    o_ref[...] = lax.dot_general(
        f_ref[...], w_ref[...], (((1,), (1,)), ((), ())),
        preferred_element_type=jnp.float32)


def _matmul(feat, W1):
    return pl.pallas_call(
        _matmul_body,
        grid=(10,),
        in_specs=[
            pl.BlockSpec((1000, D), lambda i: (i, 0)),
            pl.BlockSpec((D, D), lambda i: (0, 0)),
        ],
        out_specs=pl.BlockSpec((1000, D), lambda i: (i, 0)),
        out_shape=jax.ShapeDtypeStruct((N_NODES, D), jnp.float32),
    )(feat, W1)


def _segmax(h, src, dst):
    mesh = plsc.VectorSubcoreMesh(
        core_axis_name="c", subcore_axis_name="s",
        num_cores=NC, num_subcores=NS)

    @functools.partial(
        pl.kernel, mesh=mesh,
        out_type=jax.ShapeDtypeStruct((NPAD * D,), jnp.float32),
        scratch_types=[
            pltpu.VMEM(((RPW + 1) * D,), jnp.float32),
            pltpu.VMEM((ECH,), jnp.int32),
            pltpu.VMEM((ECH,), jnp.int32),
            pltpu.VMEM((SELCAP,), jnp.int32),
            pltpu.VMEM((SELCAP,), jnp.int32),
            [pltpu.VMEM((32, D), jnp.float32)] * 2,
            [pltpu.SemaphoreType.DMA] * 2,
        ],
        compiler_params=pltpu.CompilerParams(needs_layout_passes=False),
    )
    def k(h_hbm, src_hbm, dst_hbm, out_hbm,
          acc, srcb, dstb, sel_s, sel_d, rowsbufs, sems):
        wid = lax.axis_index("s") * NC + lax.axis_index("c")
        lo = wid * RPW
        neg = jnp.full((16,), -jnp.inf, jnp.float32)
        zero16 = jnp.zeros((16,), jnp.int32)

        def init_body(i, _):
            acc[pl.ds(i * 64, 16)] = neg
            acc[pl.ds(i * 64 + 16, 16)] = neg
            acc[pl.ds(i * 64 + 32, 16)] = neg
            acc[pl.ds(i * 64 + 48, 16)] = neg
            return 0
        lax.fori_loop(0, (RPW + 1) * D // 64, init_body, 0)

        def initsel_body(i, _):
            sel_s[pl.ds(i * 16, 16)] = zero16
            return 0
        lax.fori_loop(0, SELCAP // 16, initsel_body, 0)

        pad_s = jnp.full((16,), 0, jnp.int32) + wid
        pad_d = jnp.full((16,), RPW, jnp.int32)

        def apply_batch(rows, dlv, off):
            for e in range(16):
                base = dlv[e] * D

                @plsc.parallel_loop(0, NSL, step=1, unroll=NSL)
                def _(j):
                    sl = pl.ds(base + j * 16, 16)
                    acc[sl] = jnp.maximum(
                        acc[sl], rows[off + e, pl.ds(j * 16, 16)])

        def chunk_body(c, _):
            pltpu.sync_copy(src_hbm.at[pl.ds(c * ECH, ECH)], srcb)
            pltpu.sync_copy(dst_hbm.at[pl.ds(c * ECH, ECH)], dstb)

            def scan_body(i, cnt):
                ds_ = [dstb[pl.ds(i * 64 + u * 16, 16)] for u in range(4)]
                ss_ = [srcb[pl.ds(i * 64 + u * 16, 16)] for u in range(4)]
                dls = [d - lo for d in ds_]
                ms = [(dl >= 0) & (dl < RPW) for dl in dls]
                css = [plsc.cumsum(m.astype(jnp.int32)) for m in ms]
                c = cnt
                for u in range(4):
                    pos = c + css[u] - 1
                    plsc.store_scatter(sel_s, [pos], ss_[u], mask=ms[u])
                    plsc.store_scatter(sel_d, [pos], dls[u], mask=ms[u])
                    c = c + css[u][15]
                return c
            cnt = lax.fori_loop(0, ECH // 64, scan_body, jnp.int32(0))

            for p in range(4):
                sel_s[pl.ds(cnt + p * 16, 16)] = pad_s
                sel_d[pl.ds(cnt + p * 16, 16)] = pad_d

            nb2 = (cnt + 63) // 64
            rows0, rows1 = rowsbufs[0], rowsbufs[1]
            sem0, sem1 = sems[0], sems[1]

            pltpu.async_copy(h_hbm.at[sel_s.at[pl.ds(0, 32)]], rows0, sem0)

            def pair_body(g2, _):
                e0 = g2 * 64
                pltpu.async_copy(
                    h_hbm.at[sel_s.at[pl.ds(e0 + 32, 32)]], rows1, sem1)
                pltpu.make_async_copy(
                    h_hbm.at[sel_s.at[pl.ds(0, 32)]], rows0, sem0).wait()
                apply_batch(rows0, sel_d[pl.ds(e0, 16)], 0)
                apply_batch(rows0, sel_d[pl.ds(e0 + 16, 16)], 16)
                pltpu.async_copy(
                    h_hbm.at[sel_s.at[pl.ds(e0 + 64, 32)]], rows0, sem0)
                pltpu.make_async_copy(
                    h_hbm.at[sel_s.at[pl.ds(0, 32)]], rows1, sem1).wait()
                apply_batch(rows1, sel_d[pl.ds(e0 + 32, 16)], 0)
                apply_batch(rows1, sel_d[pl.ds(e0 + 48, 16)], 16)
                return 0
            lax.fori_loop(0, nb2, pair_body, 0)

            pltpu.make_async_copy(
                h_hbm.at[sel_s.at[pl.ds(0, 32)]], rows0, sem0).wait()
            return 0
        lax.fori_loop(0, NCHUNK, chunk_body, 0)

        zf = jnp.zeros((16,), jnp.float32)

        def fix_body(i, _):
            sls = [pl.ds(i * 64 + u * 16, 16) for u in range(4)]
            vs = [acc[sl] for sl in sls]
            for u in range(4):
                acc[sls[u]] = jnp.where(vs[u] == neg, zf, vs[u])
            return 0
        lax.fori_loop(0, RPW * D // 64, fix_body, 0)

        pltpu.sync_copy(acc.at[pl.ds(0, RPW * D)],
                        out_hbm.at[pl.ds(lo * D, RPW * D)])

    return k(h, src, dst)


def kernel(feat, edge_index, W1):
    h = _matmul(feat, W1)
    ei = edge_index.astype(jnp.int32)
    flat = _segmax(h, ei[0], ei[1])
    h_N = flat.reshape(NPAD, D)[:N_NODES]
    return jnp.concatenate([feat, h_N], axis=1)


# speedup vs baseline: 1.6711x; 1.6711x over previous
import functools

import jax
import jax.numpy as jnp
from jax import lax
from jax.experimental import pallas as pl
from jax.experimental.pallas import tpu as pltpu
from jax.experimental.pallas import tpu_sc as plsc

N_NODES = 10000
N_EDGES = 160000
D = 256

NC, NS = 2, 16
NW = NC * NS
RPW = 320
NPAD = NW * RPW
ECH = 6400
NCHUNK = N_EDGES // ECH
NSL = D // 16
SELCAP = ECH + 144


def _matmul_body(f_ref, w_ref, o_ref):
    o_ref[...] = lax.dot_general(
        f_ref[...], w_ref[...], (((1,), (1,)), ((), ())),
        preferred_element_type=jnp.float32)


def _matmul(feat, W1):
    return pl.pallas_call(
        _matmul_body,
        grid=(10,),
        in_specs=[
            pl.BlockSpec((1000, D), lambda i: (i, 0)),
            pl.BlockSpec((D, D), lambda i: (0, 0)),
        ],
        out_specs=pl.BlockSpec((1000, D), lambda i: (i, 0)),
        out_shape=jax.ShapeDtypeStruct((N_NODES, D), jnp.float32),
    )(feat, W1)


def _segmax(h, src, dst):
    mesh = plsc.VectorSubcoreMesh(
        core_axis_name="c", subcore_axis_name="s",
        num_cores=NC, num_subcores=NS)

    @functools.partial(
        pl.kernel, mesh=mesh,
        out_type=jax.ShapeDtypeStruct((NPAD * D,), jnp.float32),
        scratch_types=[
            pltpu.VMEM(((RPW + 1) * D,), jnp.float32),
            pltpu.VMEM((ECH,), jnp.int32),
            pltpu.VMEM((ECH,), jnp.int32),
            pltpu.VMEM((SELCAP,), jnp.int32),
            pltpu.VMEM((SELCAP,), jnp.int32),
            [pltpu.VMEM((16, D), jnp.float32)] * 2,
            [pltpu.SemaphoreType.DMA] * 2,
        ],
        compiler_params=pltpu.CompilerParams(needs_layout_passes=False),
    )
    def k(h_hbm, src_hbm, dst_hbm, out_hbm,
          acc, srcb, dstb, sel_s, sel_d, rowsbufs, sems):
        wid = lax.axis_index("s") * NC + lax.axis_index("c")
        lo = wid * RPW
        neg = jnp.full((16,), -jnp.inf, jnp.float32)
        zero16 = jnp.zeros((16,), jnp.int32)

        def init_body(i, _):
            acc[pl.ds(i * 64, 16)] = neg
            acc[pl.ds(i * 64 + 16, 16)] = neg
            acc[pl.ds(i * 64 + 32, 16)] = neg
            acc[pl.ds(i * 64 + 48, 16)] = neg
            return 0
        lax.fori_loop(0, (RPW + 1) * D // 64, init_body, 0)

        def initsel_body(i, _):
            sel_s[pl.ds(i * 16, 16)] = zero16
            return 0
        lax.fori_loop(0, SELCAP // 16, initsel_body, 0)

        pad_s = jnp.full((16,), 0, jnp.int32) + wid
        pad_d = jnp.full((16,), RPW, jnp.int32)

        def apply_batch(rows, dlv):
            for e in range(16):
                base = dlv[e] * D

                @plsc.parallel_loop(0, NSL, step=1, unroll=NSL)
                def _(j):
                    sl = pl.ds(base + j * 16, 16)
                    acc[sl] = jnp.maximum(acc[sl], rows[e, pl.ds(j * 16, 16)])

        def chunk_body(c, _):
            pltpu.sync_copy(src_hbm.at[pl.ds(c * ECH, ECH)], srcb)
            pltpu.sync_copy(dst_hbm.at[pl.ds(c * ECH, ECH)], dstb)

            def scan_body(i, cnt):
                ds_ = [dstb[pl.ds(i * 64 + u * 16, 16)] for u in range(4)]
                ss_ = [srcb[pl.ds(i * 64 + u * 16, 16)] for u in range(4)]
                dls = [d - lo for d in ds_]
                ms = [(dl >= 0) & (dl < RPW) for dl in dls]
                css = [plsc.cumsum(m.astype(jnp.int32)) for m in ms]
                c = cnt
                for u in range(4):
                    pos = c + css[u] - 1
                    plsc.store_scatter(sel_s, [pos], ss_[u], mask=ms[u])
                    plsc.store_scatter(sel_d, [pos], dls[u], mask=ms[u])
                    c = c + css[u][15]
                return c
            cnt = lax.fori_loop(0, ECH // 64, scan_body, jnp.int32(0))

            for p in range(2):
                sel_s[pl.ds(cnt + p * 16, 16)] = pad_s
                sel_d[pl.ds(cnt + p * 16, 16)] = pad_d

            nb2 = (cnt + 31) // 32
            rows0, rows1 = rowsbufs[0], rowsbufs[1]
            sem0, sem1 = sems[0], sems[1]

            pltpu.async_copy(h_hbm.at[sel_s[pl.ds(0, 16)]], rows0, sem0)

            def pair_body(g2, _):
                g0 = g2 * 2
                pltpu.async_copy(
                    h_hbm.at[sel_s[pl.ds(g0 * 16 + 16, 16)]], rows1, sem1)
                pltpu.make_async_copy(h_hbm.at[zero16], rows0, sem0).wait()
                apply_batch(rows0, sel_d[pl.ds(g0 * 16, 16)])
                pltpu.async_copy(
                    h_hbm.at[sel_s[pl.ds(g0 * 16 + 32, 16)]], rows0, sem0)
                pltpu.make_async_copy(h_hbm.at[zero16], rows1, sem1).wait()
                apply_batch(rows1, sel_d[pl.ds(g0 * 16 + 16, 16)])
                return 0
            lax.fori_loop(0, nb2, pair_body, 0)

            pltpu.make_async_copy(h_hbm.at[zero16], rows0, sem0).wait()
            return 0
        lax.fori_loop(0, NCHUNK, chunk_body, 0)

        zf = jnp.zeros((16,), jnp.float32)

        def fix_body(i, _):
            sls = [pl.ds(i * 64 + u * 16, 16) for u in range(4)]
            vs = [acc[sl] for sl in sls]
            for u in range(4):
                acc[sls[u]] = jnp.where(vs[u] == neg, zf, vs[u])
            return 0
        lax.fori_loop(0, RPW * D // 64, fix_body, 0)

        pltpu.sync_copy(acc.at[pl.ds(0, RPW * D)],
                        out_hbm.at[pl.ds(lo * D, RPW * D)])

    return k(h, src, dst)


def kernel(feat, edge_index, W1):
    h = _matmul(feat, W1)
    ei = edge_index.astype(jnp.int32)
    flat = _segmax(h, ei[0], ei[1])
    h_N = flat.reshape(NPAD, D)[:N_NODES]
    return jnp.concatenate([feat, h_N], axis=1)


# speedup vs baseline: 1.6861x; 1.0090x over previous
import functools

import jax
import jax.numpy as jnp
from jax import lax
from jax.experimental import pallas as pl
from jax.experimental.pallas import tpu as pltpu
from jax.experimental.pallas import tpu_sc as plsc

N_NODES = 10000
N_EDGES = 160000
D = 256

NC, NS = 2, 16
NW = NC * NS
RPW = 320
NPAD = NW * RPW
ECH = 6400
NCHUNK = N_EDGES // ECH
NSL = D // 16
SELCAP = ECH + 144


def _matmul_body(f_ref, w_ref, o_ref):
    o_ref[...] = lax.dot_general(
        f_ref[...], w_ref[...], (((1,), (1,)), ((), ())),
        preferred_element_type=jnp.float32)


def _matmul(feat, W1):
    return pl.pallas_call(
        _matmul_body,
        grid=(10,),
        in_specs=[
            pl.BlockSpec((1000, D), lambda i: (i, 0)),
            pl.BlockSpec((D, D), lambda i: (0, 0)),
        ],
        out_specs=pl.BlockSpec((1000, D), lambda i: (i, 0)),
        out_shape=jax.ShapeDtypeStruct((N_NODES, D), jnp.float32),
    )(feat, W1)


def _segmax(h, src, dst):
    mesh = plsc.VectorSubcoreMesh(
        core_axis_name="c", subcore_axis_name="s",
        num_cores=NC, num_subcores=NS)

    @functools.partial(
        pl.kernel, mesh=mesh,
        out_type=jax.ShapeDtypeStruct((NPAD * D,), jnp.float32),
        scratch_types=[
            pltpu.VMEM(((RPW + 1) * D,), jnp.float32),
            pltpu.VMEM((ECH,), jnp.int32),
            pltpu.VMEM((ECH,), jnp.int32),
            pltpu.VMEM((SELCAP,), jnp.int32),
            pltpu.VMEM((SELCAP,), jnp.int32),
            [pltpu.VMEM((16, D), jnp.float32)] * 2,
            [pltpu.SemaphoreType.DMA] * 2,
        ],
        compiler_params=pltpu.CompilerParams(needs_layout_passes=False),
    )
    def k(h_hbm, src_hbm, dst_hbm, out_hbm,
          acc, srcb, dstb, sel_s, sel_d, rowsbufs, sems):
        wid = lax.axis_index("s") * NC + lax.axis_index("c")
        lo = wid * RPW
        neg = jnp.full((16,), -jnp.inf, jnp.float32)
        zero16 = jnp.zeros((16,), jnp.int32)

        def init_body(i, _):
            acc[pl.ds(i * 64, 16)] = neg
            acc[pl.ds(i * 64 + 16, 16)] = neg
            acc[pl.ds(i * 64 + 32, 16)] = neg
            acc[pl.ds(i * 64 + 48, 16)] = neg
            return 0
        lax.fori_loop(0, (RPW + 1) * D // 64, init_body, 0)

        def initsel_body(i, _):
            sel_s[pl.ds(i * 16, 16)] = zero16
            return 0
        lax.fori_loop(0, SELCAP // 16, initsel_body, 0)

        pad_s = jnp.full((16,), 0, jnp.int32) + wid
        pad_d = jnp.full((16,), RPW, jnp.int32)

        def apply_batch(rows, dlv):
            for e in range(16):
                base = dlv[e] * D
                vals = [
                    jnp.maximum(acc[pl.ds(base + j * 16, 16)],
                                rows[e, pl.ds(j * 16, 16)])
                    for j in range(NSL)
                ]
                for j in range(NSL):
                    acc[pl.ds(base + j * 16, 16)] = vals[j]

        def chunk_body(c, _):
            pltpu.sync_copy(src_hbm.at[pl.ds(c * ECH, ECH)], srcb)
            pltpu.sync_copy(dst_hbm.at[pl.ds(c * ECH, ECH)], dstb)

            def scan_body(i, cnt):
                ds_ = [dstb[pl.ds(i * 64 + u * 16, 16)] for u in range(4)]
                ss_ = [srcb[pl.ds(i * 64 + u * 16, 16)] for u in range(4)]
                dls = [d - lo for d in ds_]
                ms = [(dl >= 0) & (dl < RPW) for dl in dls]
                css = [plsc.cumsum(m.astype(jnp.int32)) for m in ms]
                c = cnt
                for u in range(4):
                    pos = c + css[u] - 1
                    plsc.store_scatter(sel_s, [pos], ss_[u], mask=ms[u])
                    plsc.store_scatter(sel_d, [pos], dls[u], mask=ms[u])
                    c = c + css[u][15]
                return c
            cnt = lax.fori_loop(0, ECH // 64, scan_body, jnp.int32(0))

            for p in range(2):
                sel_s[pl.ds(cnt + p * 16, 16)] = pad_s
                sel_d[pl.ds(cnt + p * 16, 16)] = pad_d

            nb2 = (cnt + 31) // 32
            rows0, rows1 = rowsbufs[0], rowsbufs[1]
            sem0, sem1 = sems[0], sems[1]

            pltpu.async_copy(h_hbm.at[sel_s[pl.ds(0, 16)]], rows0, sem0)

            def pair_body(g2, _):
                g0 = g2 * 2
                pltpu.async_copy(
                    h_hbm.at[sel_s[pl.ds(g0 * 16 + 16, 16)]], rows1, sem1)
                pltpu.make_async_copy(h_hbm.at[zero16], rows0, sem0).wait()
                apply_batch(rows0, sel_d[pl.ds(g0 * 16, 16)])
                pltpu.async_copy(
                    h_hbm.at[sel_s[pl.ds(g0 * 16 + 32, 16)]], rows0, sem0)
                pltpu.make_async_copy(h_hbm.at[zero16], rows1, sem1).wait()
                apply_batch(rows1, sel_d[pl.ds(g0 * 16 + 16, 16)])
                return 0
            lax.fori_loop(0, nb2, pair_body, 0)

            pltpu.make_async_copy(h_hbm.at[zero16], rows0, sem0).wait()
            return 0
        lax.fori_loop(0, NCHUNK, chunk_body, 0)

        zf = jnp.zeros((16,), jnp.float32)

        def fix_body(i, _):
            sls = [pl.ds(i * 64 + u * 16, 16) for u in range(4)]
            vs = [acc[sl] for sl in sls]
            for u in range(4):
                acc[sls[u]] = jnp.where(vs[u] == neg, zf, vs[u])
            return 0
        lax.fori_loop(0, RPW * D // 64, fix_body, 0)

        pltpu.sync_copy(acc.at[pl.ds(0, RPW * D)],
                        out_hbm.at[pl.ds(lo * D, RPW * D)])

    return k(h, src, dst)


def kernel(feat, edge_index, W1):
    h = _matmul(feat, W1)
    ei = edge_index.astype(jnp.int32)
    flat = _segmax(h, ei[0], ei[1])
    h_N = flat.reshape(NPAD, D)[:N_NODES]
    return jnp.concatenate([feat, h_N], axis=1)
